# Initial kernel scaffold; baseline (speedup 1.0000x reference)
import jax
import jax.numpy as jnp
from jax.experimental import pallas as pl


def kernel(x, edge_index, W1, b1, W2, b2):
    raise NotImplementedError("write your pallas kernel here")


# speedup vs baseline: 12.0921x; 12.0921x over previous
import functools

import jax
import jax.numpy as jnp
from jax import lax
from jax.experimental import pallas as pl
from jax.experimental.pallas import tpu as pltpu
from jax.experimental.pallas import tpu_sc as plsc

_K = 80


def _mesh():
    return plsc.VectorSubcoreMesh(core_axis_name="c", subcore_axis_name="s")


@functools.lru_cache(maxsize=None)
def _deg_kernel(n, e, d):
    epw = e // 32
    rpw = n // 16
    assert epw % _K == 0 and n % 16 == 0

    @functools.partial(
        pl.kernel,
        out_type=jax.ShapeDtypeStruct((2 * n, d), jnp.float32),
        mesh=_mesh(),
        scratch_types=[
            pltpu.VMEM((_K,), jnp.int32),
            pltpu.VMEM((_K, d), jnp.float32),
            pltpu.VMEM_SHARED((n, d), jnp.float32),
        ],
    )
    def k(dst_hbm, zer_hbm, one_hbm, degp_hbm, dst_v, ones_v, dacc):
        cid = lax.axis_index("c")
        sid = lax.axis_index("s")
        r0 = sid * rpw
        pltpu.sync_copy(zer_hbm.at[pl.ds(r0, rpw)], dacc.at[pl.ds(r0, rpw)])
        pltpu.sync_copy(one_hbm, ones_v)
        plsc.subcore_barrier()
        base = (cid * 16 + sid) * epw

        def body(i, carry):
            off = base + i * _K
            pltpu.sync_copy(dst_hbm.at[pl.ds(off, _K)], dst_v)
            pltpu.sync_copy(ones_v, dacc.at[dst_v], add=True)
            return carry

        lax.fori_loop(0, epw // _K, body, 0)
        plsc.subcore_barrier()
        pltpu.sync_copy(dacc.at[pl.ds(r0, rpw)],
                        degp_hbm.at[pl.ds(cid * n + r0, rpw)])

    return k


@functools.lru_cache(maxsize=None)
def _edge_kernel(n, e, d):
    epw = e // 32
    rpw = n // 16
    assert epw % _K == 0 and n % 16 == 0

    @functools.partial(
        pl.kernel,
        out_type=jax.ShapeDtypeStruct((2 * n, d), jnp.float32),
        mesh=_mesh(),
        scratch_types=[
            pltpu.VMEM((_K,), jnp.int32),
            pltpu.VMEM((_K,), jnp.int32),
            pltpu.VMEM((_K, d), jnp.float32),
            pltpu.VMEM_SHARED((n, d), jnp.float32),
            pltpu.SemaphoreType.DMA,
        ],
    )
    def k(t_hbm, src_hbm, dst_hbm, zer_hbm, acc_hbm,
          src_v, dst_v, rows_v, acc, sem):
        cid = lax.axis_index("c")
        sid = lax.axis_index("s")
        r0 = sid * rpw

        @pl.when(cid == 0)
        def _():
            pltpu.sync_copy(t_hbm.at[pl.ds(r0, rpw)], acc.at[pl.ds(r0, rpw)])

        @pl.when(cid != 0)
        def _():
            pltpu.sync_copy(zer_hbm.at[pl.ds(r0, rpw)], acc.at[pl.ds(r0, rpw)])

        plsc.subcore_barrier()
        base = (cid * 16 + sid) * epw

        def body(i, carry):
            off = base + i * _K
            pltpu.sync_copy(src_hbm.at[pl.ds(off, _K)], src_v)
            pltpu.sync_copy(dst_hbm.at[pl.ds(off, _K)], dst_v)
            pltpu.async_copy(t_hbm.at[src_v], rows_v, sem).wait()
            pltpu.sync_copy(rows_v, acc.at[dst_v], add=True)
            return carry

        lax.fori_loop(0, epw // _K, body, 0)
        plsc.subcore_barrier()
        pltpu.sync_copy(acc.at[pl.ds(r0, rpw)],
                        acc_hbm.at[pl.ds(cid * n + r0, rpw)])

    return k


def _prep_body(degp_ref, x_ref, t1_ref, dinv_ref):
    deg = degp_ref[0, :, 0:1] + degp_ref[1, :, 0:1] + 1.0
    dinv = lax.rsqrt(deg)
    dinvr = jnp.broadcast_to(dinv, x_ref.shape)
    dinv_ref[...] = dinvr
    t1_ref[...] = x_ref[...] * dinvr


@functools.lru_cache(maxsize=None)
def _prep_kernel(n, d, blk):
    grid = n // blk
    return pl.pallas_call(
        _prep_body,
        grid=(grid,),
        in_specs=[
            pl.BlockSpec((2, blk, d), lambda i: (0, i, 0)),
            pl.BlockSpec((blk, d), lambda i: (i, 0)),
        ],
        out_specs=[
            pl.BlockSpec((blk, d), lambda i: (i, 0)),
            pl.BlockSpec((blk, d), lambda i: (i, 0)),
        ],
        out_shape=[
            jax.ShapeDtypeStruct((n, d), jnp.float32),
            jax.ShapeDtypeStruct((n, d), jnp.float32),
        ],
    )


def _layer_body(acc_ref, dinv_ref, w_ref, b_ref, out_ref, *, mid):
    a = (acc_ref[0] + acc_ref[1]) * dinv_ref[...]
    h = jnp.dot(a, w_ref[...], preferred_element_type=jnp.float32) + b_ref[...]
    if mid:
        h = jnp.maximum(h, 0.0) * dinv_ref[...]
    out_ref[...] = h


@functools.lru_cache(maxsize=None)
def _layer_kernel(n, d, blk, mid):
    grid = n // blk
    return pl.pallas_call(
        functools.partial(_layer_body, mid=mid),
        grid=(grid,),
        in_specs=[
            pl.BlockSpec((2, blk, d), lambda i: (0, i, 0)),
            pl.BlockSpec((blk, d), lambda i: (i, 0)),
            pl.BlockSpec((d, d), lambda i: (0, 0)),
            pl.BlockSpec((1, d), lambda i: (0, 0)),
        ],
        out_specs=pl.BlockSpec((blk, d), lambda i: (i, 0)),
        out_shape=jax.ShapeDtypeStruct((n, d), jnp.float32),
    )


def kernel(x, edge_index, W1, b1, W2, b2):
    n, d = x.shape
    e = edge_index.shape[1]
    np_ = ((n + 127) // 128) * 128
    x_p = jnp.pad(x, ((0, np_ - n), (0, 0)))
    src = edge_index[0]
    dst = edge_index[1]
    oned = jnp.ones((_K, d), jnp.float32)
    zerd = jnp.zeros((np_, d), jnp.float32)
    blk = np_ // 16

    degp = _deg_kernel(np_, e, d)(dst, zerd, oned).reshape(2, np_, d)
    t1, dinvr = _prep_kernel(np_, d, blk)(degp, x_p)
    acc1 = _edge_kernel(np_, e, d)(t1, src, dst, zerd).reshape(2, np_, d)
    t2 = _layer_kernel(np_, d, blk, True)(acc1, dinvr, W1, b1.reshape(1, d))
    acc2 = _edge_kernel(np_, e, d)(t2, src, dst, zerd).reshape(2, np_, d)
    out = _layer_kernel(np_, d, blk, False)(acc2, dinvr, W2, b2.reshape(1, d))
    return out[:n]
